# Initial kernel scaffold; baseline (speedup 1.0000x reference)
import jax
import jax.numpy as jnp
from jax.experimental import pallas as pl


def kernel(flat, cu_seqlens, W, b):
    raise NotImplementedError("write your pallas kernel here")


# speedup vs baseline: 10.9903x; 10.9903x over previous
import functools

import jax
import jax.numpy as jnp
from jax.experimental import pallas as pl
from jax.experimental.pallas import tpu as pltpu


def _body(starts_ref, ends_ref, invc_ref, x_ref, W_ref, b_ref, out_ref,
          acc_mean, acc_cls, *, blk, nblk):
    i = pl.program_id(0)
    S = acc_mean.shape[0]
    pos = jax.lax.broadcasted_iota(jnp.int32, (blk, S), 0) + i * blk
    st = starts_ref[...]
    en = ends_ref[...]
    on_mean = ((pos >= st) & (pos < en)).astype(jnp.float32)
    on_cls = (pos == st).astype(jnp.float32)
    x = x_ref[...]
    dn = (((0,), (0,)), ((), ()))
    pm = jax.lax.dot_general(on_mean, x, dn, preferred_element_type=jnp.float32)
    pc = jax.lax.dot_general(on_cls, x, dn, preferred_element_type=jnp.float32)

    @pl.when(i == 0)
    def _():
        acc_mean[...] = pm
        acc_cls[...] = pc

    @pl.when(i > 0)
    def _():
        acc_mean[...] = acc_mean[...] + pm
        acc_cls[...] = acc_cls[...] + pc

    @pl.when(i == nblk - 1)
    def _():
        mean = acc_mean[...] * invc_ref[...]
        pooled = jnp.concatenate([acc_cls[...], mean], axis=-1)
        out_ref[...] = (
            jnp.dot(pooled, W_ref[...], preferred_element_type=jnp.float32)
            + b_ref[...]
        )


def kernel(flat, cu_seqlens, W, b):
    T, D = flat.shape
    S = cu_seqlens.shape[0] - 1
    NL = W.shape[1]
    blk = 2048
    nblk = T // blk

    starts = cu_seqlens[:-1].reshape(1, S)
    ends = cu_seqlens[1:].reshape(1, S)
    counts = (cu_seqlens[1:] - cu_seqlens[:-1]).astype(jnp.float32)
    invc = (1.0 / jnp.maximum(counts, 1.0)).reshape(S, 1)
    b2 = b.reshape(1, NL)

    grid = (nblk,)
    out = pl.pallas_call(
        functools.partial(_body, blk=blk, nblk=nblk),
        grid=grid,
        in_specs=[
            pl.BlockSpec((1, S), lambda i: (0, 0)),
            pl.BlockSpec((1, S), lambda i: (0, 0)),
            pl.BlockSpec((S, 1), lambda i: (0, 0)),
            pl.BlockSpec((blk, D), lambda i: (i, 0)),
            pl.BlockSpec((2 * D, NL), lambda i: (0, 0)),
            pl.BlockSpec((1, NL), lambda i: (0, 0)),
        ],
        out_specs=pl.BlockSpec((S, NL), lambda i: (0, 0)),
        out_shape=jax.ShapeDtypeStruct((S, NL), jnp.float32),
        scratch_shapes=[
            pltpu.VMEM((S, D), jnp.float32),
            pltpu.VMEM((S, D), jnp.float32),
        ],
        compiler_params=pltpu.CompilerParams(
            dimension_semantics=("arbitrary",),
        ),
    )(starts, ends, invc, flat, W, b2)
    return out


# speedup vs baseline: 11.3189x; 1.0299x over previous
import functools

import jax
import jax.numpy as jnp
from jax.experimental import pallas as pl
from jax.experimental.pallas import tpu as pltpu


def _body(starts_ref, ends_ref, invc_ref, x_ref, W_ref, b_ref, out_ref,
          acc_mean, acc_cls, *, blk, nblk):
    i = pl.program_id(0)
    S = acc_mean.shape[0]
    pos = jax.lax.broadcasted_iota(jnp.int32, (blk, S), 0) + i * blk
    st = starts_ref[...]
    en = ends_ref[...]
    on_mean = ((pos >= st) & (pos < en)).astype(jnp.float32)
    on_cls = (pos == st).astype(jnp.float32)
    x = x_ref[...]
    dn = (((0,), (0,)), ((), ()))
    pm = jax.lax.dot_general(on_mean, x, dn, preferred_element_type=jnp.float32)
    pc = jax.lax.dot_general(on_cls, x, dn, preferred_element_type=jnp.float32)

    @pl.when(i == 0)
    def _():
        acc_mean[...] = pm
        acc_cls[...] = pc

    @pl.when(i > 0)
    def _():
        acc_mean[...] = acc_mean[...] + pm
        acc_cls[...] = acc_cls[...] + pc

    @pl.when(i == nblk - 1)
    def _():
        mean = acc_mean[...] * invc_ref[...]
        pooled = jnp.concatenate([acc_cls[...], mean], axis=-1)
        out_ref[...] = (
            jnp.dot(pooled, W_ref[...], preferred_element_type=jnp.float32)
            + b_ref[...]
        )


def kernel(flat, cu_seqlens, W, b):
    T, D = flat.shape
    S = cu_seqlens.shape[0] - 1
    NL = W.shape[1]
    blk = 4096
    nblk = T // blk

    starts = cu_seqlens[:-1].reshape(1, S)
    ends = cu_seqlens[1:].reshape(1, S)
    counts = (cu_seqlens[1:] - cu_seqlens[:-1]).astype(jnp.float32)
    invc = (1.0 / jnp.maximum(counts, 1.0)).reshape(S, 1)
    b2 = b.reshape(1, NL)

    grid = (nblk,)
    out = pl.pallas_call(
        functools.partial(_body, blk=blk, nblk=nblk),
        grid=grid,
        in_specs=[
            pl.BlockSpec((1, S), lambda i: (0, 0)),
            pl.BlockSpec((1, S), lambda i: (0, 0)),
            pl.BlockSpec((S, 1), lambda i: (0, 0)),
            pl.BlockSpec((blk, D), lambda i: (i, 0)),
            pl.BlockSpec((2 * D, NL), lambda i: (0, 0)),
            pl.BlockSpec((1, NL), lambda i: (0, 0)),
        ],
        out_specs=pl.BlockSpec((S, NL), lambda i: (0, 0)),
        out_shape=jax.ShapeDtypeStruct((S, NL), jnp.float32),
        scratch_shapes=[
            pltpu.VMEM((S, D), jnp.float32),
            pltpu.VMEM((S, D), jnp.float32),
        ],
        compiler_params=pltpu.CompilerParams(
            dimension_semantics=("arbitrary",),
        ),
    )(starts, ends, invc, flat, W, b2)
    return out
